# baseline (speedup 1.0000x reference)
import functools

import jax
import jax.numpy as jnp
from jax import lax
from jax.experimental import pallas as pl
from jax.experimental.pallas import tpu as pltpu

B, T, V, D = 16, 50, 64, 768
M = 64
POSF = 7
MAX_STEPS = 100
EPS = 1e-12
TV = T * V
ROWS = B * TV

NBUF = 8
CH = 320
TC_ = CH // V
CPB = TV // CH
NCHT = ROWS // CH
TP = 64


def _ring_kernel(x_hbm, mask_hbm, cand_hbm, lens_ref, vpids_ref, sid_ref,
                 pos_ref, wpos_ref, bpos_ref, gam_ref, bet_ref, table_ref,
                 out_ref, *scr):
    f32 = jnp.float32
    bufs = scr[0:NBUF]
    mbufs = scr[NBUF:2 * NBUF]
    cbufs = scr[2 * NBUF:3 * NBUF]
    cand_acc, vis_acc, cnt_acc = scr[3 * NBUF:3 * NBUF + 3]
    sems = scr[3 * NBUF + 3:]

    def start_all(ch, u):
        pltpu.make_async_copy(x_hbm.at[pl.ds(ch * CH, CH)], bufs[u],
                              sems[u]).start()
        pltpu.make_async_copy(mask_hbm.at[pl.ds(ch * CH, CH)], mbufs[u],
                              sems[u]).start()
        pltpu.make_async_copy(cand_hbm.at[pl.ds(ch * CH, CH)], cbufs[u],
                              sems[u]).start()

    def wait_all(ch, u):
        pltpu.make_async_copy(x_hbm.at[pl.ds(ch * CH, CH)], bufs[u],
                              sems[u]).wait()
        pltpu.make_async_copy(mask_hbm.at[pl.ds(ch * CH, CH)], mbufs[u],
                              sems[u]).wait()
        pltpu.make_async_copy(cand_hbm.at[pl.ds(ch * CH, CH)], cbufs[u],
                              sems[u]).wait()

    for u in range(NBUF):
        start_all(u, u)

    def chunk_body(i, u):
        buf = bufs[u]
        sem = sems[u]
        wait_all(i, u)
        b = i // CPB
        c = i % CPB

        x = buf[...]
        mask_col = mbufs[u][...]
        cand = cbufs[u][...]
        m_iota = jax.lax.broadcasted_iota(jnp.int32, (CH, M), 1)
        onehot = jnp.where(cand == m_iota, mask_col, 0.0)
        part = jax.lax.dot_general(
            onehot, x, (((0,), (0,)), ((), ())), preferred_element_type=f32)
        part_cnt = jnp.sum(onehot, axis=0, keepdims=True)

        masked = x * mask_col
        step_part = jnp.sum(masked.reshape(TC_, V, D), axis=1)

        vp = vpids_ref[pl.ds(b * TP, TP)]
        m_iota_t = jax.lax.broadcasted_iota(jnp.int32, (TP, M), 1)
        t_iota = jax.lax.broadcasted_iota(jnp.int32, (TP, M), 0)
        hit = (vp + 1) == m_iota_t
        tstar = jnp.max(jnp.where(hit, t_iota + 1, 0), axis=0, keepdims=True)
        tstar_col = tstar.reshape(M, 1)
        tloc = tstar_col - 1 - c * TC_
        j_iota = jax.lax.broadcasted_iota(jnp.int32, (M, TC_), 1)
        vis_c = jnp.where((tloc == j_iota) & (tstar_col > 0), 1.0, 0.0)
        vis_part = jnp.dot(vis_c, step_part, preferred_element_type=f32)

        @pl.when(c == 0)
        def _():
            cand_acc[...] = part
            vis_acc[...] = vis_part
            cnt_acc[...] = part_cnt

        @pl.when(c != 0)
        def _():
            cand_acc[...] += part
            vis_acc[...] += vis_part
            cnt_acc[...] += part_cnt

        @pl.when(c == CPB - 1)
        def _():
            lensf = jnp.maximum(lens_ref[pl.ds(b * TP, TP)], 1).astype(f32)
            unvisited = cand_acc[...] / jnp.maximum(
                cnt_acc[...].reshape(M, 1), 1.0)
            t_iota_m = jax.lax.broadcasted_iota(jnp.int32, (M, TP), 1)
            onehot_vis = ((tstar_col - 1) == t_iota_m).astype(f32)
            len_sel = jnp.dot(onehot_vis, lensf, preferred_element_type=f32)
            visited_fts = vis_acc[...] / jnp.maximum(len_sel, 1.0)
            vis_mask = tstar_col > 0
            img = jnp.where(vis_mask, visited_fts, unvisited)
            node_iota = jax.lax.broadcasted_iota(jnp.int32, (M, 1), 0)
            img = jnp.where(node_iota == 0, 0.0, img)

            sid = sid_ref[pl.ds(b * M, M)]
            s_iota = jax.lax.broadcasted_iota(jnp.int32, (M, MAX_STEPS), 1)
            onehot_step = (sid == s_iota).astype(f32)
            step_emb = jnp.dot(onehot_step, table_ref[...],
                               preferred_element_type=f32)

            h = jnp.dot(pos_ref[pl.ds(b * M, M)], wpos_ref[...],
                        preferred_element_type=f32) + bpos_ref[...]
            mu = jnp.mean(h, axis=1, keepdims=True)
            var = jnp.mean((h - mu) ** 2, axis=1, keepdims=True)
            ln = (h - mu) / jnp.sqrt(var + EPS) * gam_ref[...] + bet_ref[...]

            out_ref[pl.ds(b * M, M), :] = img + step_emb + ln

        nxt = i + NBUF

        @pl.when(nxt < NCHT)
        def _():
            start_all(nxt, u)

    def loop_body(it, carry):
        for u in range(NBUF):
            chunk_body(it * NBUF + u, u)
        return carry

    lax.fori_loop(0, NCHT // NBUF, loop_body, 0)


@jax.jit
def _encode(split_traj_embeds, split_traj_vp_lens, traj_vpids, traj_cand_vpids,
            gmap_step_ids, gmap_pos_fts, W_pos, b_pos, ln_gamma, ln_beta,
            step_table):
    x = split_traj_embeds.reshape(ROWS, D)
    lens_c = jnp.maximum(split_traj_vp_lens, 1)
    mask_flat = (jnp.arange(V)[None, None, :] < lens_c[:, :, None]).astype(
        jnp.float32).reshape(ROWS, 1)
    cand_flat = traj_cand_vpids.reshape(ROWS, 1)
    lens_pad = jnp.pad(split_traj_vp_lens, ((0, 0), (0, TP - T)),
                       constant_values=1).reshape(B * TP, 1)
    vpids_pad = jnp.pad(traj_vpids, ((0, 0), (0, TP - T)),
                        constant_values=-2).reshape(B * TP, 1)
    sid = gmap_step_ids.reshape(B * M, 1)
    pos = jnp.pad(gmap_pos_fts, ((0, 0), (0, 0), (0, 8 - POSF))).reshape(
        B * M, 8)
    wpos = jnp.pad(W_pos, ((0, 8 - POSF), (0, 0)))
    bpos = b_pos.reshape(1, D)
    gam = ln_gamma.reshape(1, D)
    bet = ln_beta.reshape(1, D)

    vmem = functools.partial(pl.BlockSpec, memory_space=pltpu.VMEM)
    out = pl.pallas_call(
        _ring_kernel,
        in_specs=[
            pl.BlockSpec(memory_space=pl.ANY),
            pl.BlockSpec(memory_space=pl.ANY),
            pl.BlockSpec(memory_space=pl.ANY),
            vmem(), vmem(), vmem(), vmem(), vmem(), vmem(),
            vmem(), vmem(), vmem(),
        ],
        out_specs=vmem(),
        out_shape=jax.ShapeDtypeStruct((B * M, D), jnp.float32),
        scratch_shapes=(
            [pltpu.VMEM((CH, D), jnp.float32) for _ in range(NBUF)]
            + [pltpu.VMEM((CH, 1), jnp.float32) for _ in range(NBUF)]
            + [pltpu.VMEM((CH, 1), jnp.int32) for _ in range(NBUF)]
            + [pltpu.VMEM((M, D), jnp.float32),
               pltpu.VMEM((M, D), jnp.float32),
               pltpu.VMEM((1, M), jnp.float32)]
            + [pltpu.SemaphoreType.DMA for _ in range(NBUF)]
        ),
    )(x, mask_flat, cand_flat, lens_pad, vpids_pad, sid, pos, wpos, bpos,
      gam, bet, step_table)
    return out.reshape(B, M, D)


def kernel(txt_embeds, txt_masks, split_traj_embeds, split_traj_vp_lens,
           traj_vpids, traj_cand_vpids, gmap_vpids, gmap_step_ids,
           gmap_pos_fts, gmap_lens, W_pos, b_pos, ln_gamma, ln_beta,
           step_table):
    return _encode(split_traj_embeds, split_traj_vp_lens, traj_vpids,
                   traj_cand_vpids, gmap_step_ids, gmap_pos_fts, W_pos, b_pos,
                   ln_gamma, ln_beta, step_table)


# speedup vs baseline: 1.1357x; 1.1357x over previous
import functools

import jax
import jax.numpy as jnp
from jax import lax
from jax.experimental import pallas as pl
from jax.experimental.pallas import tpu as pltpu

B, T, V, D = 16, 50, 64, 768
M = 64
POSF = 7
MAX_STEPS = 100
EPS = 1e-12
TV = T * V
ROWS = B * TV

NBUF = 8
CH = 640
TC_ = CH // V
CPB = TV // CH
NCHT = ROWS // CH
TP = 64


def _ring_kernel(x_hbm, mask_hbm, cand_hbm, lens_ref, vpids_ref, sid_ref,
                 pos_ref, wpos_ref, bpos_ref, gam_ref, bet_ref, table_ref,
                 out_ref, *scr):
    f32 = jnp.float32
    bufs = scr[0:NBUF]
    mbufs = scr[NBUF:2 * NBUF]
    cbufs = scr[2 * NBUF:3 * NBUF]
    cand_acc, vis_acc, cnt_acc = scr[3 * NBUF:3 * NBUF + 3]
    sems = scr[3 * NBUF + 3:]

    def start_all(ch, u):
        pltpu.make_async_copy(x_hbm.at[pl.ds(ch * CH, CH)], bufs[u],
                              sems[u]).start()
        pltpu.make_async_copy(mask_hbm.at[pl.ds(ch * CH, CH)], mbufs[u],
                              sems[u]).start()
        pltpu.make_async_copy(cand_hbm.at[pl.ds(ch * CH, CH)], cbufs[u],
                              sems[u]).start()

    def wait_all(ch, u):
        pltpu.make_async_copy(x_hbm.at[pl.ds(ch * CH, CH)], bufs[u],
                              sems[u]).wait()
        pltpu.make_async_copy(mask_hbm.at[pl.ds(ch * CH, CH)], mbufs[u],
                              sems[u]).wait()
        pltpu.make_async_copy(cand_hbm.at[pl.ds(ch * CH, CH)], cbufs[u],
                              sems[u]).wait()

    for u in range(NBUF):
        start_all(u, u)

    def chunk_body(i, u):
        buf = bufs[u]
        sem = sems[u]
        wait_all(i, u)
        b = i // CPB
        c = i % CPB

        x = buf[...]
        mask_col = mbufs[u][...]
        cand = cbufs[u][...]
        m_iota = jax.lax.broadcasted_iota(jnp.int32, (CH, M), 1)
        onehot = jnp.where(cand == m_iota, mask_col, 0.0)
        part = jax.lax.dot_general(
            onehot, x, (((0,), (0,)), ((), ())), preferred_element_type=f32)
        part_cnt = jnp.sum(onehot, axis=0, keepdims=True)

        masked = x * mask_col
        step_part = jnp.sum(masked.reshape(TC_, V, D), axis=1)

        vp = vpids_ref[pl.ds(b * TP, TP)]
        m_iota_t = jax.lax.broadcasted_iota(jnp.int32, (TP, M), 1)
        t_iota = jax.lax.broadcasted_iota(jnp.int32, (TP, M), 0)
        hit = (vp + 1) == m_iota_t
        tstar = jnp.max(jnp.where(hit, t_iota + 1, 0), axis=0, keepdims=True)
        tstar_col = tstar.reshape(M, 1)
        tloc = tstar_col - 1 - c * TC_
        j_iota = jax.lax.broadcasted_iota(jnp.int32, (M, TC_), 1)
        vis_c = jnp.where((tloc == j_iota) & (tstar_col > 0), 1.0, 0.0)
        vis_part = jnp.dot(vis_c, step_part, preferred_element_type=f32)

        @pl.when(c == 0)
        def _():
            cand_acc[...] = part
            vis_acc[...] = vis_part
            cnt_acc[...] = part_cnt

        @pl.when(c != 0)
        def _():
            cand_acc[...] += part
            vis_acc[...] += vis_part
            cnt_acc[...] += part_cnt

        @pl.when(c == CPB - 1)
        def _():
            lensf = jnp.maximum(lens_ref[pl.ds(b * TP, TP)], 1).astype(f32)
            unvisited = cand_acc[...] / jnp.maximum(
                cnt_acc[...].reshape(M, 1), 1.0)
            t_iota_m = jax.lax.broadcasted_iota(jnp.int32, (M, TP), 1)
            onehot_vis = ((tstar_col - 1) == t_iota_m).astype(f32)
            len_sel = jnp.dot(onehot_vis, lensf, preferred_element_type=f32)
            visited_fts = vis_acc[...] / jnp.maximum(len_sel, 1.0)
            vis_mask = tstar_col > 0
            img = jnp.where(vis_mask, visited_fts, unvisited)
            node_iota = jax.lax.broadcasted_iota(jnp.int32, (M, 1), 0)
            img = jnp.where(node_iota == 0, 0.0, img)

            sid = sid_ref[pl.ds(b * M, M)]
            s_iota = jax.lax.broadcasted_iota(jnp.int32, (M, MAX_STEPS), 1)
            onehot_step = (sid == s_iota).astype(f32)
            step_emb = jnp.dot(onehot_step, table_ref[...],
                               preferred_element_type=f32)

            h = jnp.dot(pos_ref[pl.ds(b * M, M)], wpos_ref[...],
                        preferred_element_type=f32) + bpos_ref[...]
            mu = jnp.mean(h, axis=1, keepdims=True)
            var = jnp.mean((h - mu) ** 2, axis=1, keepdims=True)
            ln = (h - mu) / jnp.sqrt(var + EPS) * gam_ref[...] + bet_ref[...]

            out_ref[pl.ds(b * M, M), :] = img + step_emb + ln

        nxt = i + NBUF

        @pl.when(nxt < NCHT)
        def _():
            start_all(nxt, u)

    def loop_body(it, carry):
        for u in range(NBUF):
            chunk_body(it * NBUF + u, u)
        return carry

    lax.fori_loop(0, NCHT // NBUF, loop_body, 0)


@jax.jit
def _encode(split_traj_embeds, split_traj_vp_lens, traj_vpids, traj_cand_vpids,
            gmap_step_ids, gmap_pos_fts, W_pos, b_pos, ln_gamma, ln_beta,
            step_table):
    x = split_traj_embeds.reshape(ROWS, D)
    lens_c = jnp.maximum(split_traj_vp_lens, 1)
    mask_flat = (jnp.arange(V)[None, None, :] < lens_c[:, :, None]).astype(
        jnp.float32).reshape(ROWS, 1)
    cand_flat = traj_cand_vpids.reshape(ROWS, 1)
    lens_pad = jnp.pad(split_traj_vp_lens, ((0, 0), (0, TP - T)),
                       constant_values=1).reshape(B * TP, 1)
    vpids_pad = jnp.pad(traj_vpids, ((0, 0), (0, TP - T)),
                        constant_values=-2).reshape(B * TP, 1)
    sid = gmap_step_ids.reshape(B * M, 1)
    pos = jnp.pad(gmap_pos_fts, ((0, 0), (0, 0), (0, 8 - POSF))).reshape(
        B * M, 8)
    wpos = jnp.pad(W_pos, ((0, 8 - POSF), (0, 0)))
    bpos = b_pos.reshape(1, D)
    gam = ln_gamma.reshape(1, D)
    bet = ln_beta.reshape(1, D)

    vmem = functools.partial(pl.BlockSpec, memory_space=pltpu.VMEM)
    out = pl.pallas_call(
        _ring_kernel,
        in_specs=[
            pl.BlockSpec(memory_space=pl.ANY),
            pl.BlockSpec(memory_space=pl.ANY),
            pl.BlockSpec(memory_space=pl.ANY),
            vmem(), vmem(), vmem(), vmem(), vmem(), vmem(),
            vmem(), vmem(), vmem(),
        ],
        out_specs=vmem(),
        out_shape=jax.ShapeDtypeStruct((B * M, D), jnp.float32),
        scratch_shapes=(
            [pltpu.VMEM((CH, D), jnp.float32) for _ in range(NBUF)]
            + [pltpu.VMEM((CH, 1), jnp.float32) for _ in range(NBUF)]
            + [pltpu.VMEM((CH, 1), jnp.int32) for _ in range(NBUF)]
            + [pltpu.VMEM((M, D), jnp.float32),
               pltpu.VMEM((M, D), jnp.float32),
               pltpu.VMEM((1, M), jnp.float32)]
            + [pltpu.SemaphoreType.DMA for _ in range(NBUF)]
        ),
    )(x, mask_flat, cand_flat, lens_pad, vpids_pad, sid, pos, wpos, bpos,
      gam, bet, step_table)
    return out.reshape(B, M, D)


def kernel(txt_embeds, txt_masks, split_traj_embeds, split_traj_vp_lens,
           traj_vpids, traj_cand_vpids, gmap_vpids, gmap_step_ids,
           gmap_pos_fts, gmap_lens, W_pos, b_pos, ln_gamma, ln_beta,
           step_table):
    return _encode(split_traj_embeds, split_traj_vp_lens, traj_vpids,
                   traj_cand_vpids, gmap_step_ids, gmap_pos_fts, W_pos, b_pos,
                   ln_gamma, ln_beta, step_table)


# speedup vs baseline: 1.1552x; 1.0172x over previous
import functools

import jax
import jax.numpy as jnp
from jax import lax
from jax.experimental import pallas as pl
from jax.experimental.pallas import tpu as pltpu

B, T, V, D = 16, 50, 64, 768
M = 64
POSF = 7
MAX_STEPS = 100
EPS = 1e-12
TV = T * V
ROWS = B * TV

NBUF = 4
CH = 640
TC_ = CH // V
CPB = TV // CH
NCHT = ROWS // CH
TP = 64


def _ring_kernel(x_hbm, mask_hbm, cand_hbm, lens_ref, vpids_ref, sid_ref,
                 pos_ref, wpos_ref, bpos_ref, gam_ref, bet_ref, table_ref,
                 out_ref, *scr):
    f32 = jnp.float32
    bufs = scr[0:NBUF]
    mbufs = scr[NBUF:2 * NBUF]
    cbufs = scr[2 * NBUF:3 * NBUF]
    cand_acc, vis_acc, cnt_acc = scr[3 * NBUF:3 * NBUF + 3]
    sems = scr[3 * NBUF + 3:]

    def start_all(ch, u):
        pltpu.make_async_copy(x_hbm.at[pl.ds(ch * CH, CH)], bufs[u],
                              sems[u]).start()
        pltpu.make_async_copy(mask_hbm.at[pl.ds(ch * CH, CH)], mbufs[u],
                              sems[u]).start()
        pltpu.make_async_copy(cand_hbm.at[pl.ds(ch * CH, CH)], cbufs[u],
                              sems[u]).start()

    def wait_all(ch, u):
        pltpu.make_async_copy(x_hbm.at[pl.ds(ch * CH, CH)], bufs[u],
                              sems[u]).wait()
        pltpu.make_async_copy(mask_hbm.at[pl.ds(ch * CH, CH)], mbufs[u],
                              sems[u]).wait()
        pltpu.make_async_copy(cand_hbm.at[pl.ds(ch * CH, CH)], cbufs[u],
                              sems[u]).wait()

    for u in range(NBUF):
        start_all(u, u)

    def chunk_body(i, u):
        buf = bufs[u]
        sem = sems[u]
        wait_all(i, u)
        b = i // CPB
        c = i % CPB

        x = buf[...]
        mask_col = mbufs[u][...]
        cand = cbufs[u][...]
        m_iota = jax.lax.broadcasted_iota(jnp.int32, (CH, M), 1)
        onehot = jnp.where(cand == m_iota, mask_col, 0.0)
        part = jax.lax.dot_general(
            onehot, x, (((0,), (0,)), ((), ())), preferred_element_type=f32)
        part_cnt = jnp.sum(onehot, axis=0, keepdims=True)

        masked = x * mask_col
        step_part = jnp.sum(masked.reshape(TC_, V, D), axis=1)

        vp = vpids_ref[pl.ds(b * TP, TP)]
        m_iota_t = jax.lax.broadcasted_iota(jnp.int32, (TP, M), 1)
        t_iota = jax.lax.broadcasted_iota(jnp.int32, (TP, M), 0)
        hit = (vp + 1) == m_iota_t
        tstar = jnp.max(jnp.where(hit, t_iota + 1, 0), axis=0, keepdims=True)
        tstar_col = tstar.reshape(M, 1)
        tloc = tstar_col - 1 - c * TC_
        j_iota = jax.lax.broadcasted_iota(jnp.int32, (M, TC_), 1)
        vis_c = jnp.where((tloc == j_iota) & (tstar_col > 0), 1.0, 0.0)
        vis_part = jnp.dot(vis_c, step_part, preferred_element_type=f32)

        @pl.when(c == 0)
        def _():
            cand_acc[...] = part
            vis_acc[...] = vis_part
            cnt_acc[...] = part_cnt

        @pl.when(c != 0)
        def _():
            cand_acc[...] += part
            vis_acc[...] += vis_part
            cnt_acc[...] += part_cnt

        @pl.when(c == CPB - 1)
        def _():
            lensf = jnp.maximum(lens_ref[pl.ds(b * TP, TP)], 1).astype(f32)
            unvisited = cand_acc[...] / jnp.maximum(
                cnt_acc[...].reshape(M, 1), 1.0)
            t_iota_m = jax.lax.broadcasted_iota(jnp.int32, (M, TP), 1)
            onehot_vis = ((tstar_col - 1) == t_iota_m).astype(f32)
            len_sel = jnp.dot(onehot_vis, lensf, preferred_element_type=f32)
            visited_fts = vis_acc[...] / jnp.maximum(len_sel, 1.0)
            vis_mask = tstar_col > 0
            img = jnp.where(vis_mask, visited_fts, unvisited)
            node_iota = jax.lax.broadcasted_iota(jnp.int32, (M, 1), 0)
            img = jnp.where(node_iota == 0, 0.0, img)

            sid = sid_ref[pl.ds(b * M, M)]
            s_iota = jax.lax.broadcasted_iota(jnp.int32, (M, MAX_STEPS), 1)
            onehot_step = (sid == s_iota).astype(f32)
            step_emb = jnp.dot(onehot_step, table_ref[...],
                               preferred_element_type=f32)

            h = jnp.dot(pos_ref[pl.ds(b * M, M)], wpos_ref[...],
                        preferred_element_type=f32) + bpos_ref[...]
            mu = jnp.mean(h, axis=1, keepdims=True)
            var = jnp.mean((h - mu) ** 2, axis=1, keepdims=True)
            ln = (h - mu) / jnp.sqrt(var + EPS) * gam_ref[...] + bet_ref[...]

            out_ref[pl.ds(b * M, M), :] = img + step_emb + ln

        nxt = i + NBUF

        @pl.when(nxt < NCHT)
        def _():
            start_all(nxt, u)

    def loop_body(it, carry):
        for u in range(NBUF):
            chunk_body(it * NBUF + u, u)
        return carry

    lax.fori_loop(0, NCHT // NBUF, loop_body, 0)


@jax.jit
def _encode(split_traj_embeds, split_traj_vp_lens, traj_vpids, traj_cand_vpids,
            gmap_step_ids, gmap_pos_fts, W_pos, b_pos, ln_gamma, ln_beta,
            step_table):
    x = split_traj_embeds.reshape(ROWS, D)
    lens_c = jnp.maximum(split_traj_vp_lens, 1)
    mask_flat = (jnp.arange(V)[None, None, :] < lens_c[:, :, None]).astype(
        jnp.float32).reshape(ROWS, 1)
    cand_flat = traj_cand_vpids.reshape(ROWS, 1)
    lens_pad = jnp.pad(split_traj_vp_lens, ((0, 0), (0, TP - T)),
                       constant_values=1).reshape(B * TP, 1)
    vpids_pad = jnp.pad(traj_vpids, ((0, 0), (0, TP - T)),
                        constant_values=-2).reshape(B * TP, 1)
    sid = gmap_step_ids.reshape(B * M, 1)
    pos = jnp.pad(gmap_pos_fts, ((0, 0), (0, 0), (0, 8 - POSF))).reshape(
        B * M, 8)
    wpos = jnp.pad(W_pos, ((0, 8 - POSF), (0, 0)))
    bpos = b_pos.reshape(1, D)
    gam = ln_gamma.reshape(1, D)
    bet = ln_beta.reshape(1, D)

    vmem = functools.partial(pl.BlockSpec, memory_space=pltpu.VMEM)
    out = pl.pallas_call(
        _ring_kernel,
        in_specs=[
            pl.BlockSpec(memory_space=pl.ANY),
            pl.BlockSpec(memory_space=pl.ANY),
            pl.BlockSpec(memory_space=pl.ANY),
            vmem(), vmem(), vmem(), vmem(), vmem(), vmem(),
            vmem(), vmem(), vmem(),
        ],
        out_specs=vmem(),
        out_shape=jax.ShapeDtypeStruct((B * M, D), jnp.float32),
        scratch_shapes=(
            [pltpu.VMEM((CH, D), jnp.float32) for _ in range(NBUF)]
            + [pltpu.VMEM((CH, 1), jnp.float32) for _ in range(NBUF)]
            + [pltpu.VMEM((CH, 1), jnp.int32) for _ in range(NBUF)]
            + [pltpu.VMEM((M, D), jnp.float32),
               pltpu.VMEM((M, D), jnp.float32),
               pltpu.VMEM((1, M), jnp.float32)]
            + [pltpu.SemaphoreType.DMA for _ in range(NBUF)]
        ),
    )(x, mask_flat, cand_flat, lens_pad, vpids_pad, sid, pos, wpos, bpos,
      gam, bet, step_table)
    return out.reshape(B, M, D)


def kernel(txt_embeds, txt_masks, split_traj_embeds, split_traj_vp_lens,
           traj_vpids, traj_cand_vpids, gmap_vpids, gmap_step_ids,
           gmap_pos_fts, gmap_lens, W_pos, b_pos, ln_gamma, ln_beta,
           step_table):
    return _encode(split_traj_embeds, split_traj_vp_lens, traj_vpids,
                   traj_cand_vpids, gmap_step_ids, gmap_pos_fts, W_pos, b_pos,
                   ln_gamma, ln_beta, step_table)
